# baseline (speedup 1.0000x reference)
import functools

import numpy as np
import jax
import jax.numpy as jnp
from jax import lax
from jax.experimental import pallas as pl
from jax.experimental.pallas import tpu as pltpu


def _build_rhs(w, K, W):
    w3 = w.reshape(K, 3, 3)
    eyes = np.stack([np.eye(W, k=1), np.eye(W, k=0), np.eye(W, k=-1)])
    E = jnp.asarray(eyes, jnp.float32)
    R = jnp.einsum("kjx,xci->jcki", w3, E)
    return R.reshape(3 * W, K * W).astype(jnp.bfloat16)


def _lhs_parts(img_ref, B, H, W, with_sums):
    zrow = jnp.zeros((1, W), jnp.float32)
    rgb = []
    parts = []
    sum_rows = []
    for bi in range(B):
        r = img_ref[bi, 0]
        g = img_ref[bi, 1]
        b = img_ref[bi, 2]
        rgb.append((r, g, b))
        lr = jnp.log(r + 1e-7)
        lg = jnp.log(g + 1e-7)
        lb = jnp.log(b + 1e-7)
        for d in (lr - lg, lg - lb):
            up = jnp.concatenate([zrow, d[:H - 1]], axis=0)
            dn = jnp.concatenate([d[1:], zrow], axis=0)
            parts.append(jnp.concatenate([up, d, dn], axis=1))
            if with_sums:
                cs = jnp.sum(d, axis=0, keepdims=True)
                sum_rows.append(jnp.concatenate(
                    [cs - d[H - 1:H], cs, cs - d[0:1]], axis=1))
    return rgb, parts, sum_rows


def _stats_kernel(img_ref, g_ref, cs_ref, *, B, H, W):
    @pl.when(pl.program_id(0) == 0)
    def _init():
        g_ref[...] = jnp.zeros_like(g_ref)
        cs_ref[...] = jnp.zeros_like(cs_ref)

    _, parts, sum_rows = _lhs_parts(img_ref, B, H, W, with_sums=True)
    row_rg = sum_rows[0]
    row_gb = sum_rows[1]
    for bi in range(1, B):
        row_rg = row_rg + sum_rows[2 * bi]
        row_gb = row_gb + sum_rows[2 * bi + 1]
    l_rg = jnp.concatenate(parts[0::2], axis=0).astype(jnp.bfloat16)
    l_gb = jnp.concatenate(parts[1::2], axis=0).astype(jnp.bfloat16)
    dims = (((0,), (0,)), ((), ()))
    g_ref[0] += lax.dot_general(l_rg, l_rg, dims,
                                preferred_element_type=jnp.float32)
    g_ref[1] += lax.dot_general(l_gb, l_gb, dims,
                                preferred_element_type=jnp.float32)
    g_ref[2] += lax.dot_general(l_rg, l_gb, dims,
                                preferred_element_type=jnp.float32)
    cs_ref[0:1] += row_rg
    cs_ref[1:2] += row_gb


def _apply_kernel(bn_ref, r_ref, img_ref, out_ref, *, B, K, H, W):
    rgb, parts, _ = _lhs_parts(img_ref, B, H, W, with_sums=False)
    L = jnp.concatenate(parts, axis=0).astype(jnp.bfloat16)
    P = lax.dot_general(L, r_ref[...],
                        dimension_numbers=(((1,), (0,)), ((), ())),
                        preferred_element_type=jnp.float32)
    for bi in range(B):
        r, g, b = rgb[bi]
        zr = r == 0.0
        zg = g == 0.0
        zb = b == 0.0
        for k in range(K):
            c_rg = P[(2 * bi) * H:(2 * bi) * H + H, k * W:(k + 1) * W]
            c_gb = P[(2 * bi + 1) * H:(2 * bi + 1) * H + H, k * W:(k + 1) * W]
            c_rb = c_rg + c_gb
            for gi, (c, zm) in enumerate(((c_rg, zr), (c_gb, zg), (c_rb, zb))):
                ch = gi * K + k
                y = c * bn_ref[0, ch] + bn_ref[1, ch]
                out_ref[bi, ch] = jnp.where(zm, 0.0, y)


def kernel(img, filt):
    N, C, H, W = img.shape
    assert C == 3
    K = filt.shape[0]
    ntaps = filt.shape[2] * filt.shape[3]

    img_f = img.astype(jnp.float32)
    w = filt.reshape(K, ntaps).astype(jnp.float32)
    w = w - jnp.mean(w, axis=1, keepdims=True)
    rhs = _build_rhs(w, K, W)

    B = 8 if N % 8 == 0 else 1
    Bs = 32 if N % 32 == 0 else B
    rhs_spec = pl.BlockSpec((3 * W, K * W), lambda n: (0, 0))
    img_spec = pl.BlockSpec((B, 3, H, W), lambda n: (n, 0, 0, 0))
    vmem_limit = 64 * 1024 * 1024

    g3, cs = pl.pallas_call(
        functools.partial(_stats_kernel, B=Bs, H=H, W=W),
        out_shape=(jax.ShapeDtypeStruct((3, 3 * W, 3 * W), jnp.float32),
                   jax.ShapeDtypeStruct((2, 3 * W), jnp.float32)),
        grid=(N // Bs,),
        in_specs=[pl.BlockSpec((Bs, 3, H, W), lambda n: (n, 0, 0, 0))],
        out_specs=(pl.BlockSpec((3, 3 * W, 3 * W), lambda n: (0, 0, 0)),
                   pl.BlockSpec((2, 3 * W), lambda n: (0, 0))),
        compiler_params=pltpu.CompilerParams(
            dimension_semantics=("arbitrary",),
            vmem_limit_bytes=vmem_limit),
    )(img_f)

    m_np = np.zeros((3, 3, W, W), np.float32)
    for a in range(3):
        for b in range(3):
            x_lo = max(0, 1 - a, 1 - b)
            x_hi = min(W - 1, W - a, W - b)
            for x in range(x_lo, x_hi + 1):
                m_np[a, b, x + a - 1, x + b - 1] = 1.0
    m_ab = jnp.asarray(m_np)
    ma_np = np.zeros((3, W), np.float32)
    for a in range(3):
        ma_np[a, max(0, a - 1):W + min(0, a - 1)] = 1.0
    m_a = jnp.asarray(ma_np)

    gr = g3.reshape(3, 3, W, 3, W).transpose(0, 1, 3, 2, 4)
    g9raw = lax.dot_general(
        gr.reshape(27, W * W), m_ab.reshape(9, W * W),
        dimension_numbers=(((1,), (1,)), ((), ())),
        preferred_element_type=jnp.float32,
        precision=lax.Precision.DEFAULT)
    g9 = (g9raw.reshape(3, 3, 3, 3, 3)
          .transpose(0, 1, 3, 2, 4).reshape(3, 9, 9))
    ssq3 = jnp.einsum("kt,ptu,ku->pk", w, g9, w)
    s9 = jnp.einsum("gjc,ac->gja", cs.reshape(2, 3, W), m_a).reshape(2, 9)
    sums2 = jnp.einsum("kt,gt->gk", w, s9)
    cnt = jnp.float32(N * H * W)
    sums = jnp.concatenate([sums2[0], sums2[1], sums2[0] + sums2[1]])
    sumsq = jnp.concatenate([ssq3[0], ssq3[1],
                             ssq3[0] + ssq3[1] + 2.0 * ssq3[2]])
    mean = sums / cnt
    var = jnp.maximum(sumsq / cnt - mean * mean, 0.0)
    scale = 0.01 * lax.rsqrt(var + 1e-5)
    bn = jnp.stack([scale, -mean * scale], axis=0)

    out = pl.pallas_call(
        functools.partial(_apply_kernel, B=B, K=K, H=H, W=W),
        out_shape=jax.ShapeDtypeStruct((N, 3 * K, H, W), jnp.float32),
        grid=(N // B,),
        in_specs=[pl.BlockSpec(memory_space=pltpu.SMEM), rhs_spec, img_spec],
        out_specs=pl.BlockSpec((B, 3 * K, H, W), lambda n: (n, 0, 0, 0)),
        compiler_params=pltpu.CompilerParams(
            dimension_semantics=("parallel",),
            vmem_limit_bytes=vmem_limit),
    )(bn, rhs, img_f)
    return out


# speedup vs baseline: 1.0054x; 1.0054x over previous
import functools

import numpy as np
import jax
import jax.numpy as jnp
from jax import lax
from jax.experimental import pallas as pl
from jax.experimental.pallas import tpu as pltpu


def _build_rhs(w, K, W):
    w3 = w.reshape(K, 3, 3)
    eyes = np.stack([np.eye(W, k=1), np.eye(W, k=0), np.eye(W, k=-1)])
    E = jnp.asarray(eyes, jnp.float32)
    R = jnp.einsum("kjx,xci->jcki", w3, E)
    return R.reshape(3 * W, K * W).astype(jnp.bfloat16)


def _lhs_parts(img_ref, B, H, W, with_sums):
    zrow = jnp.zeros((1, W), jnp.float32)
    rgb = []
    parts = []
    sum_rows = []
    for bi in range(B):
        r = img_ref[bi, 0]
        g = img_ref[bi, 1]
        b = img_ref[bi, 2]
        rgb.append((r, g, b))
        lr = jnp.log(r + 1e-7)
        lg = jnp.log(g + 1e-7)
        lb = jnp.log(b + 1e-7)
        for d in (lr - lg, lg - lb):
            up = jnp.concatenate([zrow, d[:H - 1]], axis=0)
            dn = jnp.concatenate([d[1:], zrow], axis=0)
            parts.append(jnp.concatenate([up, d, dn], axis=1))
            if with_sums:
                cs = jnp.sum(d, axis=0, keepdims=True)
                sum_rows.append(jnp.concatenate(
                    [cs - d[H - 1:H], cs, cs - d[0:1]], axis=1))
    return rgb, parts, sum_rows


def _stats_kernel(img_ref, g_ref, cs_ref, *, B, H, W):
    @pl.when(pl.program_id(0) == 0)
    def _init():
        g_ref[...] = jnp.zeros_like(g_ref)
        cs_ref[...] = jnp.zeros_like(cs_ref)

    _, parts, sum_rows = _lhs_parts(img_ref, B, H, W, with_sums=True)
    row_rg = sum_rows[0]
    row_gb = sum_rows[1]
    for bi in range(1, B):
        row_rg = row_rg + sum_rows[2 * bi]
        row_gb = row_gb + sum_rows[2 * bi + 1]
    l_rg = jnp.concatenate(parts[0::2], axis=0).astype(jnp.bfloat16)
    l_gb = jnp.concatenate(parts[1::2], axis=0).astype(jnp.bfloat16)
    dims = (((0,), (0,)), ((), ()))
    g_ref[0] += lax.dot_general(l_rg, l_rg, dims,
                                preferred_element_type=jnp.float32)
    g_ref[1] += lax.dot_general(l_gb, l_gb, dims,
                                preferred_element_type=jnp.float32)
    g_ref[2] += lax.dot_general(l_rg, l_gb, dims,
                                preferred_element_type=jnp.float32)
    cs_ref[0:1] += row_rg
    cs_ref[1:2] += row_gb


def _apply_kernel(bn_ref, r_ref, img_ref, out_ref, *, B, K, H, W):
    rgb, parts, _ = _lhs_parts(img_ref, B, H, W, with_sums=False)
    L = jnp.concatenate(parts, axis=0).astype(jnp.bfloat16)
    P = lax.dot_general(L, r_ref[...],
                        dimension_numbers=(((1,), (0,)), ((), ())),
                        preferred_element_type=jnp.float32)
    for bi in range(B):
        r, g, b = rgb[bi]
        zr = r == 0.0
        zg = g == 0.0
        zb = b == 0.0
        for k in range(K):
            c_rg = P[(2 * bi) * H:(2 * bi) * H + H, k * W:(k + 1) * W]
            c_gb = P[(2 * bi + 1) * H:(2 * bi + 1) * H + H, k * W:(k + 1) * W]
            c_rb = c_rg + c_gb
            for gi, (c, zm) in enumerate(((c_rg, zr), (c_gb, zg), (c_rb, zb))):
                ch = gi * K + k
                y = c * bn_ref[0, ch] + bn_ref[1, ch]
                out_ref[bi, ch] = jnp.where(zm, 0.0, y)


def kernel(img, filt):
    N, C, H, W = img.shape
    assert C == 3
    K = filt.shape[0]
    ntaps = filt.shape[2] * filt.shape[3]

    img_f = img.astype(jnp.float32)
    w = filt.reshape(K, ntaps).astype(jnp.float32)
    w = w - jnp.mean(w, axis=1, keepdims=True)
    rhs = _build_rhs(w, K, W)

    B = 8 if N % 8 == 0 else 1
    Bs = 32 if N % 32 == 0 else B
    rhs_spec = pl.BlockSpec((3 * W, K * W), lambda n: (0, 0))
    img_spec = pl.BlockSpec((B, 3, H, W), lambda n: (n, 0, 0, 0))
    vmem_limit = 64 * 1024 * 1024

    g3, cs = pl.pallas_call(
        functools.partial(_stats_kernel, B=Bs, H=H, W=W),
        out_shape=(jax.ShapeDtypeStruct((3, 3 * W, 3 * W), jnp.float32),
                   jax.ShapeDtypeStruct((2, 3 * W), jnp.float32)),
        grid=(N // Bs,),
        in_specs=[pl.BlockSpec((Bs, 3, H, W), lambda n: (n, 0, 0, 0))],
        out_specs=(pl.BlockSpec((3, 3 * W, 3 * W), lambda n: (0, 0, 0)),
                   pl.BlockSpec((2, 3 * W), lambda n: (0, 0))),
        compiler_params=pltpu.CompilerParams(
            dimension_semantics=("arbitrary",),
            vmem_limit_bytes=vmem_limit),
    )(img_f)

    m_np = np.zeros((3, 3, W, W), np.float32)
    for a in range(3):
        for b in range(3):
            x_lo = max(0, 1 - a, 1 - b)
            x_hi = min(W - 1, W - a, W - b)
            for x in range(x_lo, x_hi + 1):
                m_np[a, b, x + a - 1, x + b - 1] = 1.0
    m_ab = jnp.asarray(m_np)
    ma_np = np.zeros((3, W), np.float32)
    for a in range(3):
        ma_np[a, max(0, a - 1):W + min(0, a - 1)] = 1.0
    m_a = jnp.asarray(ma_np)

    gr = g3.reshape(3, 3, W, 3, W).transpose(0, 1, 3, 2, 4)
    g9raw = gr.reshape(27, W * W) @ m_ab.reshape(9, W * W).T
    g9 = (g9raw.reshape(3, 3, 3, 3, 3)
          .transpose(0, 1, 3, 2, 4).reshape(3, 9, 9))
    ssq3 = jnp.einsum("kt,ptu,ku->pk", w, g9, w)
    s9 = jnp.einsum("gjc,ac->gja", cs.reshape(2, 3, W), m_a).reshape(2, 9)
    sums2 = jnp.einsum("kt,gt->gk", w, s9)
    cnt = jnp.float32(N * H * W)
    sums = jnp.concatenate([sums2[0], sums2[1], sums2[0] + sums2[1]])
    sumsq = jnp.concatenate([ssq3[0], ssq3[1],
                             ssq3[0] + ssq3[1] + 2.0 * ssq3[2]])
    mean = sums / cnt
    var = jnp.maximum(sumsq / cnt - mean * mean, 0.0)
    scale = 0.01 * lax.rsqrt(var + 1e-5)
    bn = jnp.stack([scale, -mean * scale], axis=0)

    out = pl.pallas_call(
        functools.partial(_apply_kernel, B=B, K=K, H=H, W=W),
        out_shape=jax.ShapeDtypeStruct((N, 3 * K, H, W), jnp.float32),
        grid=(N // B,),
        in_specs=[pl.BlockSpec(memory_space=pltpu.SMEM), rhs_spec, img_spec],
        out_specs=pl.BlockSpec((B, 3 * K, H, W), lambda n: (n, 0, 0, 0)),
        compiler_params=pltpu.CompilerParams(
            dimension_semantics=("parallel",),
            vmem_limit_bytes=vmem_limit),
    )(bn, rhs, img_f)
    return out
